# baseline (speedup 1.0000x reference)
import functools

import jax
import jax.numpy as jnp
from jax import lax
from jax.experimental import pallas as pl
from jax.experimental.pallas import tpu as pltpu
from jax.experimental.pallas import tpu_sc as plsc

N = 4096
U = 512
V = U + 1
BOS = 1
EOS = 2

NC = 2
NS = 16
NW = NC * NS
RW = N // NW
C = 64
NCH = RW // C
UNR = 8

_mesh = plsc.VectorSubcoreMesh(core_axis_name="c", subcore_axis_name="s")


@functools.partial(
    pl.kernel,
    out_type=[
        jax.ShapeDtypeStruct((N * V,), jnp.int32),
        jax.ShapeDtypeStruct((N * V,), jnp.int32),
    ],
    mesh=_mesh,
    scratch_types=[
        pltpu.VMEM((C * U,), jnp.int32),
        pltpu.VMEM((C * V,), jnp.int32),
        pltpu.VMEM((C * V,), jnp.int32),
        pltpu.VMEM((RW,), jnp.int32),
    ],
    compiler_params=pltpu.CompilerParams(needs_layout_passes=False),
)
def _seq_extract(y_hbm, ly_hbm, ypad_hbm, tgt_hbm, ybuf, pbuf, tbuf, lybuf):
    wid = lax.axis_index("s") * NC + lax.axis_index("c")
    base = wid * RW
    iota = lax.iota(jnp.int32, 16)

    pltpu.sync_copy(ly_hbm.at[pl.ds(base, RW)], lybuf)

    for ch in range(NCH):
        r0 = base + ch * C
        pltpu.sync_copy(y_hbm.at[pl.ds(r0 * U, C * U)], ybuf)

        def body(k, _):
            for u in range(UNR):
                off = k * (UNR * 16) + u * 16
                v = ybuf[pl.ds(off, 16)]
                e = off + iota
                dst = e + lax.shift_right_logical(e, 9)
                plsc.store_scatter(tbuf, [dst], v)
                plsc.store_scatter(pbuf, [dst + 1], v)
            return 0

        lax.fori_loop(0, (C * U) // (UNR * 16), body, 0)

        for g in range(C // 16):
            rows = (iota + g * 16) * V
            plsc.store_scatter(pbuf, [rows], jnp.full((16,), BOS, jnp.int32))
            plsc.store_scatter(tbuf, [rows + U], jnp.zeros((16,), jnp.int32))
            lyv = lybuf[pl.ds(ch * C + g * 16, 16)]
            plsc.store_scatter(tbuf, [rows + lyv], jnp.full((16,), EOS, jnp.int32))

        pltpu.sync_copy(pbuf, ypad_hbm.at[pl.ds(r0 * V, C * V)])
        pltpu.sync_copy(tbuf, tgt_hbm.at[pl.ds(r0 * V, C * V)])


def kernel(y, ly):
    ypad_flat, tgt_flat = _seq_extract(y.reshape(N * U), ly)
    return ypad_flat.reshape(N, V), tgt_flat.reshape(N, V)
